# baseline (speedup 1.0000x reference)
import functools

import jax
import jax.numpy as jnp
from jax import lax
from jax.experimental import pallas as pl
from jax.experimental.pallas import tpu as pltpu
from jax.experimental.pallas import tpu_sc as plsc

NC = 2
NS = 16
NW = NC * NS
LANES = 16


def _pack2(x):
    k = x.shape[1] // 2
    lo = jax.lax.bitcast_convert_type(
        x[:, :k].astype(jnp.bfloat16), jnp.uint16).astype(jnp.uint32)
    hi = jax.lax.bitcast_convert_type(
        x[:, k:].astype(jnp.bfloat16), jnp.uint16).astype(jnp.uint32)
    return jax.lax.bitcast_convert_type(lo | (hi << 16), jnp.int32)


def _unpack2(p):
    pu = jax.lax.bitcast_convert_type(p, jnp.uint32)
    lo = jax.lax.bitcast_convert_type(
        (pu & 0xFFFF).astype(jnp.uint16), jnp.bfloat16).astype(jnp.float32)
    hi = jax.lax.bitcast_convert_type(
        (pu >> 16).astype(jnp.uint16), jnp.bfloat16).astype(jnp.float32)
    return lo, hi


def _node_proj_body(h_ref, xs_ref, wsh_ref, wsx_ref, wth_ref, wtx_ref,
                    s_ref, t_ref):
    h = h_ref[...]
    xs = xs_ref[...]
    s = (jnp.dot(h, wsh_ref[...], preferred_element_type=jnp.float32)
         + jnp.dot(xs, wsx_ref[...], preferred_element_type=jnp.float32))
    t = (jnp.dot(h, wth_ref[...], preferred_element_type=jnp.float32)
         + jnp.dot(xs, wtx_ref[...], preferred_element_type=jnp.float32))
    s_ref[...] = _pack2(s)
    t_ref[...] = _pack2(t)


def _edge_mlp_body(ps_ref, pt_ref, ef_ref, w1e0_ref, w1e1_ref,
                   w2t0_ref, w2t1_ref, g_ref):
    s_lo, s_hi = _unpack2(ps_ref[...])
    t_lo, t_hi = _unpack2(pt_ref[...])
    ef = ef_ref[...]
    h0 = jnp.maximum(
        s_lo + t_lo
        + jnp.dot(ef, w1e0_ref[...], preferred_element_type=jnp.float32),
        0.0)
    h1 = jnp.maximum(
        s_hi + t_hi
        + jnp.dot(ef, w1e1_ref[...], preferred_element_type=jnp.float32),
        0.0)
    raw = (jnp.dot(h0, w2t0_ref[...], preferred_element_type=jnp.float32)
           + jnp.dot(h1, w2t1_ref[...], preferred_element_type=jnp.float32))
    g_ref[...] = jax.nn.sigmoid(raw)


def _out_proj_body(a0_ref, a1_ref, w_ref, o_ref):
    o_ref[...] = jnp.dot(a0_ref[...] + a1_ref[...], w_ref[...],
                         preferred_element_type=jnp.float32)


def _make_gather_add(E, width, chunk):
    n_chunks_total = E // chunk
    full_rounds = n_chunks_total // NW
    extra = n_chunks_total - full_rounds * NW
    NB = 3
    n_groups = full_rounds // NB
    remg = full_rounds - n_groups * NB
    mesh = plsc.VectorSubcoreMesh(core_axis_name="c", subcore_axis_name="s")
    wp = width // 2

    @functools.partial(
        pl.kernel, mesh=mesh,
        out_type=[
            jax.ShapeDtypeStruct((E, wp), jnp.int32),
            jax.ShapeDtypeStruct((E, wp), jnp.int32),
        ],
        scratch_types=(
            [pltpu.VMEM((chunk,), jnp.int32)] * (2 * NB)
            + [pltpu.VMEM((chunk, wp), jnp.int32)] * (2 * NB)
            + [pltpu.SemaphoreType.DMA] * (3 * NB)
        ))
    def gather_add(s_hbm, t_hbm, src_hbm, dst_hbm, ps_hbm, pt_hbm, *scr):
        sidxs = scr[0:NB]
        didxs = scr[NB:2 * NB]
        srows = scr[2 * NB:3 * NB]
        trows = scr[3 * NB:4 * NB]
        sis = scr[4 * NB:5 * NB]
        sss = scr[5 * NB:6 * NB]
        sts = scr[6 * NB:7 * NB]
        wid = lax.axis_index("s") * NC + lax.axis_index("c")

        def cbase(k):
            return pl.multiple_of((wid + k * NW) * chunk, 8)

        def idx_fetch(k, b):
            base = cbase(k)
            return (
                pltpu.async_copy(src_hbm.at[pl.ds(base, chunk)],
                                 sidxs[b], sis[b]),
                pltpu.async_copy(dst_hbm.at[pl.ds(base, chunk)],
                                 didxs[b], sis[b]),
            )

        def gath(b):
            return (
                pltpu.async_copy(s_hbm.at[sidxs[b]], srows[b], sss[b]),
                pltpu.async_copy(t_hbm.at[didxs[b]], trows[b], sts[b]),
            )

        def out(k, b):
            base = cbase(k)
            return (
                pltpu.async_copy(srows[b], ps_hbm.at[pl.ds(base, chunk)],
                                 sss[b]),
                pltpu.async_copy(trows[b], pt_hbm.at[pl.ds(base, chunk)],
                                 sts[b]),
            )

        def group(k0, m):
            ids = [idx_fetch(k0 + q, q) for q in range(m)]
            gs = []
            for q in range(m):
                for cp in ids[q]:
                    cp.wait()
                gs.append(gath(q))
            outs = []
            for q in range(m):
                for cp in gs[q]:
                    cp.wait()
                outs.append(out(k0 + q, q))
            for o in outs:
                for cp in o:
                    cp.wait()

        def body(i, carry):
            group(NB * i, NB)
            return carry

        lax.fori_loop(0, n_groups, body, 0)
        if remg:
            group(n_groups * NB, remg)
        if extra:
            @pl.when(wid < extra)
            def _():
                group(full_rounds, 1)

    return gather_add


def _make_scatter(N, E, D, H, chunk):
    epw = E // NW
    n_chunks = epw // chunk
    hd = D // H
    assert hd == LANES
    mesh = plsc.VectorSubcoreMesh(core_axis_name="c", subcore_axis_name="s")

    NB = 4
    n_groups = n_chunks // NB
    rem = n_chunks - n_groups * NB

    @functools.partial(
        pl.kernel, mesh=mesh,
        out_type=jax.ShapeDtypeStruct((NC, N, D), jnp.float32),
        scratch_types=(
            [pltpu.VMEM((chunk,), jnp.int32)] * (2 * NB)
            + [pltpu.VMEM((chunk, D), jnp.float32)] * NB
            + [pltpu.VMEM((chunk * H,), jnp.float32)] * NB
            + [pltpu.VMEM_SHARED((N, D), jnp.float32)]
            + [pltpu.SemaphoreType.DMA] * (4 * NB)
        ))
    def scatter(h_hbm, gates_hbm, src_hbm, dst_hbm, zeros_hbm, agg_hbm,
                *scr):
        sidxs = scr[0:NB]
        didxs = scr[NB:2 * NB]
        hrows = scr[2 * NB:3 * NB]
        grows = scr[3 * NB:4 * NB]
        agg_sp = scr[4 * NB]
        sis = scr[4 * NB + 1:5 * NB + 1]
        shs = scr[5 * NB + 1:6 * NB + 1]
        sgs = scr[6 * NB + 1:7 * NB + 1]
        sc_s = scr[7 * NB + 1:8 * NB + 1]
        cid = lax.axis_index("c")
        sid = lax.axis_index("s")
        wid = sid * NC + cid
        base0 = wid * epw

        @pl.when(sid == 0)
        def _():
            pltpu.sync_copy(zeros_hbm, agg_sp)

        plsc.subcore_barrier()

        def idx_fetch(g, b):
            base = pl.multiple_of(base0 + g * chunk, 8)
            return (
                pltpu.async_copy(src_hbm.at[pl.ds(base, chunk)],
                                 sidxs[b], sis[b]),
                pltpu.async_copy(dst_hbm.at[pl.ds(base, chunk)],
                                 didxs[b], sis[b]),
            )

        def gath(g, b):
            gbase = pl.multiple_of((base0 + g * chunk) * H, 8)
            return (
                pltpu.async_copy(h_hbm.at[sidxs[b]], hrows[b], shs[b]),
                pltpu.async_copy(gates_hbm.at[pl.ds(gbase, chunk * H)],
                                 grows[b], sgs[b]),
            )

        def compute_scat(b):
            hrow, grow = hrows[b], grows[b]

            def epair(p, c2):
                gv = grow[pl.ds(p * 2 * H, 2 * H)]
                for b2 in range(2):
                    e = 2 * p + b2
                    for j in range(H):
                        idxc = jnp.full((LANES,), b2 * H + j, jnp.int32)
                        gsp = gv.at[idxc].get(mode="promise_in_bounds")
                        sl = pl.ds(j * LANES, LANES)
                        hrow[e, sl] = hrow[e, sl] * gsp
                return c2

            lax.fori_loop(0, chunk // 2, epair, 0)
            return pltpu.async_copy(hrow, agg_sp.at[didxs[b]], sc_s[b],
                                    add=True)

        def group(c0, m):
            ids = [idx_fetch(c0 + q, q) for q in range(m)]
            gs = []
            for q in range(m):
                for cp in ids[q]:
                    cp.wait()
                gs.append(gath(c0 + q, q))
            scs = []
            for q in range(m):
                for cp in gs[q]:
                    cp.wait()
                scs.append(compute_scat(q))
            for cp in scs:
                cp.wait()

        def body(i, carry):
            group(NB * i, NB)
            return carry

        lax.fori_loop(0, n_groups, body, 0)
        if rem:
            group(n_groups * NB, rem)

        plsc.subcore_barrier()

        @pl.when(sid == 0)
        def _():
            pltpu.sync_copy(agg_sp, agg_hbm.at[cid])

    return scatter


def kernel(h, x_s, edge_index, edge_features, W1, W2, Wout):
    N, D = h.shape
    SF = x_s.shape[1]
    E = edge_index.shape[1]
    H, width = W2.shape

    src = edge_index[1]
    dest = edge_index[0]

    w_sh = W1[:, :D].T
    w_th = W1[:, D:2 * D].T
    w_sx = W1[:, 2 * D:2 * D + SF].T
    w_tx = W1[:, 2 * D + SF:2 * D + 2 * SF].T
    w_e = W1[:, 2 * D + 2 * SF:].T
    w2t = W2.T
    wout_t = Wout.T

    bn = 1000
    grid_n = N // bn
    S, T = pl.pallas_call(
        _node_proj_body,
        grid=(grid_n,),
        in_specs=[
            pl.BlockSpec((bn, D), lambda i: (i, 0)),
            pl.BlockSpec((bn, SF), lambda i: (i, 0)),
            pl.BlockSpec((D, width), lambda i: (0, 0)),
            pl.BlockSpec((SF, width), lambda i: (0, 0)),
            pl.BlockSpec((D, width), lambda i: (0, 0)),
            pl.BlockSpec((SF, width), lambda i: (0, 0)),
        ],
        out_specs=[
            pl.BlockSpec((bn, width // 2), lambda i: (i, 0)),
            pl.BlockSpec((bn, width // 2), lambda i: (i, 0)),
        ],
        out_shape=[
            jax.ShapeDtypeStruct((N, width // 2), jnp.int32),
            jax.ShapeDtypeStruct((N, width // 2), jnp.int32),
        ],
    )(h, x_s, w_sh, w_sx, w_th, w_tx)

    chunk = 80
    pre_s, pre_t = _make_gather_add(E, width, 128)(S, T, src, dest)

    be = 2000
    grid_e = E // be
    EF = edge_features.shape[1]
    wh = width // 2
    gates = pl.pallas_call(
        _edge_mlp_body,
        grid=(grid_e,),
        in_specs=[
            pl.BlockSpec((be, wh), lambda i: (i, 0)),
            pl.BlockSpec((be, wh), lambda i: (i, 0)),
            pl.BlockSpec((be, EF), lambda i: (i, 0)),
            pl.BlockSpec((EF, wh), lambda i: (0, 0)),
            pl.BlockSpec((EF, wh), lambda i: (0, 0)),
            pl.BlockSpec((wh, H), lambda i: (0, 0)),
            pl.BlockSpec((wh, H), lambda i: (0, 0)),
        ],
        out_specs=pl.BlockSpec((be, H), lambda i: (i, 0)),
        out_shape=jax.ShapeDtypeStruct((E, H), jnp.float32),
    )(pre_s, pre_t, edge_features, w_e[:, :wh], w_e[:, wh:],
      w2t[:wh], w2t[wh:])

    zeros = jnp.zeros((N, D), jnp.float32)
    agg = _make_scatter(N, E, D, H, chunk)(
        h, gates.reshape(E * H), src, dest, zeros)

    projected = pl.pallas_call(
        _out_proj_body,
        grid=(grid_n,),
        in_specs=[
            pl.BlockSpec((bn, D), lambda i: (i, 0)),
            pl.BlockSpec((bn, D), lambda i: (i, 0)),
            pl.BlockSpec((D, D), lambda i: (0, 0)),
        ],
        out_specs=pl.BlockSpec((bn, D), lambda i: (i, 0)),
        out_shape=jax.ShapeDtypeStruct((N, D), jnp.float32),
    )(agg[0], agg[1], wout_t)

    return projected, gates


# speedup vs baseline: 1.0503x; 1.0503x over previous
import functools

import jax
import jax.numpy as jnp
from jax import lax
from jax.experimental import pallas as pl
from jax.experimental.pallas import tpu as pltpu
from jax.experimental.pallas import tpu_sc as plsc

NC = 2
NS = 16
NW = NC * NS
LANES = 16


def _pack2(x):
    k = x.shape[1] // 2
    lo = jax.lax.bitcast_convert_type(
        x[:, :k].astype(jnp.bfloat16), jnp.uint16).astype(jnp.uint32)
    hi = jax.lax.bitcast_convert_type(
        x[:, k:].astype(jnp.bfloat16), jnp.uint16).astype(jnp.uint32)
    return jax.lax.bitcast_convert_type(lo | (hi << 16), jnp.int32)


def _unpack2(p):
    pu = jax.lax.bitcast_convert_type(p, jnp.uint32)
    lo = jax.lax.bitcast_convert_type(
        (pu & 0xFFFF).astype(jnp.uint16), jnp.bfloat16).astype(jnp.float32)
    hi = jax.lax.bitcast_convert_type(
        (pu >> 16).astype(jnp.uint16), jnp.bfloat16).astype(jnp.float32)
    return lo, hi


def _node_proj_body(h_ref, xs_ref, wsh_ref, wsx_ref, wth_ref, wtx_ref,
                    s_ref, t_ref):
    h = h_ref[...]
    xs = xs_ref[...]
    s = (jnp.dot(h, wsh_ref[...], preferred_element_type=jnp.float32)
         + jnp.dot(xs, wsx_ref[...], preferred_element_type=jnp.float32))
    t = (jnp.dot(h, wth_ref[...], preferred_element_type=jnp.float32)
         + jnp.dot(xs, wtx_ref[...], preferred_element_type=jnp.float32))
    s_ref[...] = _pack2(s)
    t_ref[...] = _pack2(t)


def _edge_mlp_body(ps_ref, pt_ref, ef_ref, w1e0_ref, w1e1_ref,
                   w2t0_ref, w2t1_ref, g_ref):
    s_lo, s_hi = _unpack2(ps_ref[...])
    t_lo, t_hi = _unpack2(pt_ref[...])
    ef = ef_ref[...]
    h0 = jnp.maximum(
        s_lo + t_lo
        + jnp.dot(ef, w1e0_ref[...], preferred_element_type=jnp.float32),
        0.0)
    h1 = jnp.maximum(
        s_hi + t_hi
        + jnp.dot(ef, w1e1_ref[...], preferred_element_type=jnp.float32),
        0.0)
    raw = (jnp.dot(h0, w2t0_ref[...], preferred_element_type=jnp.float32)
           + jnp.dot(h1, w2t1_ref[...], preferred_element_type=jnp.float32))
    g_ref[...] = jax.nn.sigmoid(raw)


def _out_proj_body(a0_ref, a1_ref, b0_ref, b1_ref, w_ref, o_ref):
    acc = (a0_ref[...] + a1_ref[...]) + (b0_ref[...] + b1_ref[...])
    o_ref[...] = jnp.dot(acc, w_ref[...],
                         preferred_element_type=jnp.float32)


def _make_gather_add(E, width, chunk):
    n_chunks_total = E // chunk
    full_rounds = n_chunks_total // NW
    extra = n_chunks_total - full_rounds * NW
    NB = 3
    n_groups = full_rounds // NB
    remg = full_rounds - n_groups * NB
    mesh = plsc.VectorSubcoreMesh(core_axis_name="c", subcore_axis_name="s")
    wp = width // 2

    @functools.partial(
        pl.kernel, mesh=mesh,
        out_type=[
            jax.ShapeDtypeStruct((E, wp), jnp.int32),
            jax.ShapeDtypeStruct((E, wp), jnp.int32),
        ],
        scratch_types=(
            [pltpu.VMEM((chunk,), jnp.int32)] * (2 * NB)
            + [pltpu.VMEM((chunk, wp), jnp.int32)] * (2 * NB)
            + [pltpu.SemaphoreType.DMA] * (3 * NB)
        ))
    def gather_add(s_hbm, t_hbm, src_hbm, dst_hbm, ps_hbm, pt_hbm, *scr):
        sidxs = scr[0:NB]
        didxs = scr[NB:2 * NB]
        srows = scr[2 * NB:3 * NB]
        trows = scr[3 * NB:4 * NB]
        sis = scr[4 * NB:5 * NB]
        sss = scr[5 * NB:6 * NB]
        sts = scr[6 * NB:7 * NB]
        wid = lax.axis_index("s") * NC + lax.axis_index("c")

        def cbase(k):
            return pl.multiple_of((wid + k * NW) * chunk, 8)

        def idx_fetch(k, b):
            base = cbase(k)
            return (
                pltpu.async_copy(src_hbm.at[pl.ds(base, chunk)],
                                 sidxs[b], sis[b]),
                pltpu.async_copy(dst_hbm.at[pl.ds(base, chunk)],
                                 didxs[b], sis[b]),
            )

        def gath(b):
            return (
                pltpu.async_copy(s_hbm.at[sidxs[b]], srows[b], sss[b]),
                pltpu.async_copy(t_hbm.at[didxs[b]], trows[b], sts[b]),
            )

        def out(k, b):
            base = cbase(k)
            return (
                pltpu.async_copy(srows[b], ps_hbm.at[pl.ds(base, chunk)],
                                 sss[b]),
                pltpu.async_copy(trows[b], pt_hbm.at[pl.ds(base, chunk)],
                                 sts[b]),
            )

        def group(k0, m):
            ids = [idx_fetch(k0 + q, q) for q in range(m)]
            gs = []
            for q in range(m):
                for cp in ids[q]:
                    cp.wait()
                gs.append(gath(q))
            outs = []
            for q in range(m):
                for cp in gs[q]:
                    cp.wait()
                outs.append(out(k0 + q, q))
            for o in outs:
                for cp in o:
                    cp.wait()

        def body(i, carry):
            group(NB * i, NB)
            return carry

        lax.fori_loop(0, n_groups, body, 0)
        if remg:
            group(n_groups * NB, remg)
        if extra:
            @pl.when(wid < extra)
            def _():
                group(full_rounds, 1)

    return gather_add


def _make_scatter(N, E, D, H, chunk):
    n_chunks_total = E // chunk
    full_rounds = n_chunks_total // NW
    extra = n_chunks_total - full_rounds * NW
    hd = D // H
    assert hd == LANES
    mesh = plsc.VectorSubcoreMesh(core_axis_name="c", subcore_axis_name="s")

    NB = 2
    n_groups = full_rounds // NB
    remg = full_rounds - n_groups * NB

    @functools.partial(
        pl.kernel, mesh=mesh,
        out_type=jax.ShapeDtypeStruct((NC, N, D), jnp.float32),
        scratch_types=(
            [pltpu.VMEM((chunk,), jnp.int32)] * (2 * NB)
            + [pltpu.VMEM((chunk, D), jnp.float32)] * NB
            + [pltpu.VMEM((chunk * H,), jnp.float32)] * NB
            + [pltpu.VMEM_SHARED((N, D), jnp.float32)]
            + [pltpu.SemaphoreType.DMA] * (4 * NB)
        ))
    def scatter(h_hbm, gates_hbm, src_hbm, dst_hbm, zeros_hbm, agg_hbm,
                *scr):
        sidxs = scr[0:NB]
        didxs = scr[NB:2 * NB]
        hrows = scr[2 * NB:3 * NB]
        grows = scr[3 * NB:4 * NB]
        agg_sp = scr[4 * NB]
        sis = scr[4 * NB + 1:5 * NB + 1]
        shs = scr[5 * NB + 1:6 * NB + 1]
        sgs = scr[6 * NB + 1:7 * NB + 1]
        sc_s = scr[7 * NB + 1:8 * NB + 1]
        cid = lax.axis_index("c")
        sid = lax.axis_index("s")
        wid = sid * NC + cid

        @pl.when(sid == 0)
        def _():
            pltpu.sync_copy(zeros_hbm, agg_sp)

        plsc.subcore_barrier()

        def cbase(k):
            return pl.multiple_of((wid + k * NW) * chunk, 8)

        def idx_fetch(k, b):
            base = cbase(k)
            return (
                pltpu.async_copy(src_hbm.at[pl.ds(base, chunk)],
                                 sidxs[b], sis[b]),
                pltpu.async_copy(dst_hbm.at[pl.ds(base, chunk)],
                                 didxs[b], sis[b]),
            )

        def gath(k, b):
            gbase = pl.multiple_of(cbase(k) * H, 8)
            return (
                pltpu.async_copy(h_hbm.at[sidxs[b]], hrows[b], shs[b]),
                pltpu.async_copy(gates_hbm.at[pl.ds(gbase, chunk * H)],
                                 grows[b], sgs[b]),
            )

        def compute_scat(b):
            hrow, grow = hrows[b], grows[b]

            def epair(p, c2):
                gv = grow[pl.ds(p * 2 * H, 2 * H)]
                for b2 in range(2):
                    e = 2 * p + b2
                    for j in range(H):
                        idxc = jnp.full((LANES,), b2 * H + j, jnp.int32)
                        gsp = gv.at[idxc].get(mode="promise_in_bounds")
                        sl = pl.ds(j * LANES, LANES)
                        hrow[e, sl] = hrow[e, sl] * gsp
                return c2

            lax.fori_loop(0, chunk // 2, epair, 0)
            return pltpu.async_copy(hrow, agg_sp.at[didxs[b]], sc_s[b],
                                    add=True)

        def group(k0, m):
            ids = [idx_fetch(k0 + q, q) for q in range(m)]
            gs = []
            for q in range(m):
                for cp in ids[q]:
                    cp.wait()
                gs.append(gath(k0 + q, q))
            scs = []
            for q in range(m):
                for cp in gs[q]:
                    cp.wait()
                scs.append(compute_scat(q))
            for cp in scs:
                cp.wait()

        def body(i, carry):
            group(NB * i, NB)
            return carry

        lax.fori_loop(0, n_groups, body, 0)
        if remg:
            group(n_groups * NB, remg)
        if extra:
            @pl.when(wid < extra)
            def _():
                group(full_rounds, 1)

        plsc.subcore_barrier()

        @pl.when(sid == 0)
        def _():
            pltpu.sync_copy(agg_sp, agg_hbm.at[cid])

    return scatter


def kernel(h, x_s, edge_index, edge_features, W1, W2, Wout):
    N, D = h.shape
    SF = x_s.shape[1]
    E = edge_index.shape[1]
    H, width = W2.shape

    src = edge_index[1]
    dest = edge_index[0]

    w_sh = W1[:, :D].T
    w_th = W1[:, D:2 * D].T
    w_sx = W1[:, 2 * D:2 * D + SF].T
    w_tx = W1[:, 2 * D + SF:2 * D + 2 * SF].T
    w_e = W1[:, 2 * D + 2 * SF:].T
    w2t = W2.T
    wout_t = Wout.T

    bn = 1000
    grid_n = N // bn
    S, T = pl.pallas_call(
        _node_proj_body,
        grid=(grid_n,),
        in_specs=[
            pl.BlockSpec((bn, D), lambda i: (i, 0)),
            pl.BlockSpec((bn, SF), lambda i: (i, 0)),
            pl.BlockSpec((D, width), lambda i: (0, 0)),
            pl.BlockSpec((SF, width), lambda i: (0, 0)),
            pl.BlockSpec((D, width), lambda i: (0, 0)),
            pl.BlockSpec((SF, width), lambda i: (0, 0)),
        ],
        out_specs=[
            pl.BlockSpec((bn, width // 2), lambda i: (i, 0)),
            pl.BlockSpec((bn, width // 2), lambda i: (i, 0)),
        ],
        out_shape=[
            jax.ShapeDtypeStruct((N, width // 2), jnp.int32),
            jax.ShapeDtypeStruct((N, width // 2), jnp.int32),
        ],
    )(h, x_s, w_sh, w_sx, w_th, w_tx)

    EF = edge_features.shape[1]
    wh = width // 2
    be = 2000
    Eh = E // 2
    zeros = jnp.zeros((N, D), jnp.float32)
    gather_k = _make_gather_add(Eh, width, 128)
    scatter_k = _make_scatter(N, Eh, D, H, 128)

    def edge_mlp(pre_s, pre_t, ef):
        return pl.pallas_call(
            _edge_mlp_body,
            grid=(Eh // be,),
            in_specs=[
                pl.BlockSpec((be, wh), lambda i: (i, 0)),
                pl.BlockSpec((be, wh), lambda i: (i, 0)),
                pl.BlockSpec((be, EF), lambda i: (i, 0)),
                pl.BlockSpec((EF, wh), lambda i: (0, 0)),
                pl.BlockSpec((EF, wh), lambda i: (0, 0)),
                pl.BlockSpec((wh, H), lambda i: (0, 0)),
                pl.BlockSpec((wh, H), lambda i: (0, 0)),
            ],
            out_specs=pl.BlockSpec((be, H), lambda i: (i, 0)),
            out_shape=jax.ShapeDtypeStruct((Eh, H), jnp.float32),
        )(pre_s, pre_t, ef, w_e[:, :wh], w_e[:, wh:], w2t[:wh], w2t[wh:])

    halves = []
    for lo in (0, Eh):
        halves.append({
            "src": lax.slice(src, (lo,), (lo + Eh,)),
            "dest": lax.slice(dest, (lo,), (lo + Eh,)),
            "ef": lax.slice(edge_features, (lo, 0), (lo + Eh, EF)),
        })

    pre_a = gather_k(S, T, halves[0]["src"], halves[0]["dest"])
    pre_b = gather_k(S, T, halves[1]["src"], halves[1]["dest"])

    gates_a = edge_mlp(pre_a[0], pre_a[1], halves[0]["ef"])
    gates_b = edge_mlp(pre_b[0], pre_b[1], halves[1]["ef"])

    agg_a = scatter_k(h, gates_a.reshape(Eh * H), halves[0]["src"],
                      halves[0]["dest"], zeros)
    agg_b = scatter_k(h, gates_b.reshape(Eh * H), halves[1]["src"],
                      halves[1]["dest"], zeros)

    gates = jnp.concatenate([gates_a, gates_b], axis=0)

    projected = pl.pallas_call(
        _out_proj_body,
        grid=(grid_n,),
        in_specs=[
            pl.BlockSpec((bn, D), lambda i: (i, 0)),
            pl.BlockSpec((bn, D), lambda i: (i, 0)),
            pl.BlockSpec((bn, D), lambda i: (i, 0)),
            pl.BlockSpec((bn, D), lambda i: (i, 0)),
            pl.BlockSpec((D, D), lambda i: (0, 0)),
        ],
        out_specs=pl.BlockSpec((bn, D), lambda i: (i, 0)),
        out_shape=jax.ShapeDtypeStruct((N, D), jnp.float32),
    )(agg_a[0], agg_a[1], agg_b[0], agg_b[1], wout_t)

    return projected, gates


# speedup vs baseline: 1.0687x; 1.0175x over previous
import functools

import jax
import jax.numpy as jnp
from jax import lax
from jax.experimental import pallas as pl
from jax.experimental.pallas import tpu as pltpu
from jax.experimental.pallas import tpu_sc as plsc

NC = 2
NS = 16
NW = NC * NS
LANES = 16


def _pack2(x):
    k = x.shape[1] // 2
    lo = jax.lax.bitcast_convert_type(
        x[:, :k].astype(jnp.bfloat16), jnp.uint16).astype(jnp.uint32)
    hi = jax.lax.bitcast_convert_type(
        x[:, k:].astype(jnp.bfloat16), jnp.uint16).astype(jnp.uint32)
    return jax.lax.bitcast_convert_type(lo | (hi << 16), jnp.int32)


def _unpack2(p):
    pu = jax.lax.bitcast_convert_type(p, jnp.uint32)
    lo = jax.lax.bitcast_convert_type(
        (pu & 0xFFFF).astype(jnp.uint16), jnp.bfloat16).astype(jnp.float32)
    hi = jax.lax.bitcast_convert_type(
        (pu >> 16).astype(jnp.uint16), jnp.bfloat16).astype(jnp.float32)
    return lo, hi


def _node_proj_body(h_ref, xs_ref, wsh_ref, wsx_ref, wth_ref, wtx_ref,
                    s_ref, t_ref):
    h = h_ref[...]
    xs = xs_ref[...]
    s = (jnp.dot(h, wsh_ref[...], preferred_element_type=jnp.float32)
         + jnp.dot(xs, wsx_ref[...], preferred_element_type=jnp.float32))
    t = (jnp.dot(h, wth_ref[...], preferred_element_type=jnp.float32)
         + jnp.dot(xs, wtx_ref[...], preferred_element_type=jnp.float32))
    s_ref[...] = _pack2(s)
    t_ref[...] = _pack2(t)


def _edge_mlp_body(ps_ref, pt_ref, ef_ref, w1e0_ref, w1e1_ref,
                   w2t0_ref, w2t1_ref, g_ref):
    s_lo, s_hi = _unpack2(ps_ref[...])
    t_lo, t_hi = _unpack2(pt_ref[...])
    ef = ef_ref[...]
    h0 = jnp.maximum(
        s_lo + t_lo
        + jnp.dot(ef, w1e0_ref[...], preferred_element_type=jnp.float32),
        0.0)
    h1 = jnp.maximum(
        s_hi + t_hi
        + jnp.dot(ef, w1e1_ref[...], preferred_element_type=jnp.float32),
        0.0)
    raw = (jnp.dot(h0, w2t0_ref[...], preferred_element_type=jnp.float32)
           + jnp.dot(h1, w2t1_ref[...], preferred_element_type=jnp.float32))
    g_ref[...] = jax.nn.sigmoid(raw)


def _out_proj_body(a0_ref, a1_ref, b0_ref, b1_ref, w_ref, o_ref):
    acc = (a0_ref[...] + a1_ref[...]) + (b0_ref[...] + b1_ref[...])
    o_ref[...] = jnp.dot(acc, w_ref[...],
                         preferred_element_type=jnp.float32)


def _make_gather_add(E, width, chunk):
    n_chunks_total = E // chunk
    full_rounds = n_chunks_total // NW
    extra = n_chunks_total - full_rounds * NW
    NB = 3
    n_groups = full_rounds // NB
    remg = full_rounds - n_groups * NB
    mesh = plsc.VectorSubcoreMesh(core_axis_name="c", subcore_axis_name="s")
    wp = width // 2

    @functools.partial(
        pl.kernel, mesh=mesh,
        out_type=[
            jax.ShapeDtypeStruct((E, wp), jnp.int32),
            jax.ShapeDtypeStruct((E, wp), jnp.int32),
        ],
        scratch_types=(
            [pltpu.VMEM((chunk,), jnp.int32)] * (2 * NB)
            + [pltpu.VMEM((chunk, wp), jnp.int32)] * (2 * NB)
            + [pltpu.SemaphoreType.DMA] * (3 * NB)
        ))
    def gather_add(s_hbm, t_hbm, src_hbm, dst_hbm, ps_hbm, pt_hbm, *scr):
        sidxs = scr[0:NB]
        didxs = scr[NB:2 * NB]
        srows = scr[2 * NB:3 * NB]
        trows = scr[3 * NB:4 * NB]
        sis = scr[4 * NB:5 * NB]
        sss = scr[5 * NB:6 * NB]
        sts = scr[6 * NB:7 * NB]
        wid = lax.axis_index("s") * NC + lax.axis_index("c")

        def cbase(k):
            return pl.multiple_of((wid + k * NW) * chunk, 8)

        def idx_fetch(k, b):
            base = cbase(k)
            return (
                pltpu.async_copy(src_hbm.at[pl.ds(base, chunk)],
                                 sidxs[b], sis[b]),
                pltpu.async_copy(dst_hbm.at[pl.ds(base, chunk)],
                                 didxs[b], sis[b]),
            )

        def gath(b):
            return (
                pltpu.async_copy(s_hbm.at[sidxs[b]], srows[b], sss[b]),
                pltpu.async_copy(t_hbm.at[didxs[b]], trows[b], sts[b]),
            )

        def out(k, b):
            base = cbase(k)
            return (
                pltpu.async_copy(srows[b], ps_hbm.at[pl.ds(base, chunk)],
                                 sss[b]),
                pltpu.async_copy(trows[b], pt_hbm.at[pl.ds(base, chunk)],
                                 sts[b]),
            )

        def group(k0, m):
            ids = [idx_fetch(k0 + q, q) for q in range(m)]
            gs = []
            for q in range(m):
                for cp in ids[q]:
                    cp.wait()
                gs.append(gath(q))
            outs = []
            for q in range(m):
                for cp in gs[q]:
                    cp.wait()
                outs.append(out(k0 + q, q))
            for o in outs:
                for cp in o:
                    cp.wait()

        def body(i, carry):
            group(NB * i, NB)
            return carry

        lax.fori_loop(0, n_groups, body, 0)
        if remg:
            group(n_groups * NB, remg)
        if extra:
            @pl.when(wid < extra)
            def _():
                group(full_rounds, 1)

    return gather_add


def _make_scatter(N, E, D, H, chunk):
    n_chunks_total = E // chunk
    full_rounds = n_chunks_total // NW
    extra = n_chunks_total - full_rounds * NW
    hd = D // H
    assert hd == LANES
    mesh = plsc.VectorSubcoreMesh(core_axis_name="c", subcore_axis_name="s")

    NB = 4
    n_groups = full_rounds // NB
    remg = full_rounds - n_groups * NB

    @functools.partial(
        pl.kernel, mesh=mesh,
        out_type=jax.ShapeDtypeStruct((NC, N, D), jnp.float32),
        scratch_types=(
            [pltpu.VMEM((chunk,), jnp.int32)] * (2 * NB)
            + [pltpu.VMEM((chunk, D), jnp.float32)] * NB
            + [pltpu.VMEM((chunk * H,), jnp.float32)] * NB
            + [pltpu.VMEM_SHARED((N, D), jnp.float32)]
            + [pltpu.SemaphoreType.DMA] * (4 * NB)
        ))
    def scatter(h_hbm, gates_hbm, src_hbm, dst_hbm, zeros_hbm, agg_hbm,
                *scr):
        sidxs = scr[0:NB]
        didxs = scr[NB:2 * NB]
        hrows = scr[2 * NB:3 * NB]
        grows = scr[3 * NB:4 * NB]
        agg_sp = scr[4 * NB]
        sis = scr[4 * NB + 1:5 * NB + 1]
        shs = scr[5 * NB + 1:6 * NB + 1]
        sgs = scr[6 * NB + 1:7 * NB + 1]
        sc_s = scr[7 * NB + 1:8 * NB + 1]
        cid = lax.axis_index("c")
        sid = lax.axis_index("s")
        wid = sid * NC + cid

        @pl.when(sid == 0)
        def _():
            pltpu.sync_copy(zeros_hbm, agg_sp)

        plsc.subcore_barrier()

        def cbase(k):
            return pl.multiple_of((wid + k * NW) * chunk, 8)

        def idx_fetch(k, b):
            base = cbase(k)
            return (
                pltpu.async_copy(src_hbm.at[pl.ds(base, chunk)],
                                 sidxs[b], sis[b]),
                pltpu.async_copy(dst_hbm.at[pl.ds(base, chunk)],
                                 didxs[b], sis[b]),
            )

        def gath(k, b):
            gbase = pl.multiple_of(cbase(k) * H, 8)
            return (
                pltpu.async_copy(h_hbm.at[sidxs[b]], hrows[b], shs[b]),
                pltpu.async_copy(gates_hbm.at[pl.ds(gbase, chunk * H)],
                                 grows[b], sgs[b]),
            )

        def compute_scat(b):
            hrow, grow = hrows[b], grows[b]

            def epair(p, c2):
                gv = grow[pl.ds(p * 2 * H, 2 * H)]
                for b2 in range(2):
                    e = 2 * p + b2
                    for j in range(H):
                        idxc = jnp.full((LANES,), b2 * H + j, jnp.int32)
                        gsp = gv.at[idxc].get(mode="promise_in_bounds")
                        sl = pl.ds(j * LANES, LANES)
                        hrow[e, sl] = hrow[e, sl] * gsp
                return c2

            lax.fori_loop(0, chunk // 2, epair, 0)
            return pltpu.async_copy(hrow, agg_sp.at[didxs[b]], sc_s[b],
                                    add=True)

        def group(k0, m):
            ids = [idx_fetch(k0 + q, q) for q in range(m)]
            gs = []
            for q in range(m):
                for cp in ids[q]:
                    cp.wait()
                gs.append(gath(k0 + q, q))
            scs = []
            for q in range(m):
                for cp in gs[q]:
                    cp.wait()
                scs.append(compute_scat(q))
            for cp in scs:
                cp.wait()

        def body(i, carry):
            group(NB * i, NB)
            return carry

        lax.fori_loop(0, n_groups, body, 0)
        if remg:
            group(n_groups * NB, remg)
        if extra:
            @pl.when(wid < extra)
            def _():
                group(full_rounds, 1)

        plsc.subcore_barrier()

        @pl.when(sid == 0)
        def _():
            pltpu.sync_copy(agg_sp, agg_hbm.at[cid])

    return scatter


def kernel(h, x_s, edge_index, edge_features, W1, W2, Wout):
    N, D = h.shape
    SF = x_s.shape[1]
    E = edge_index.shape[1]
    H, width = W2.shape

    src = edge_index[1]
    dest = edge_index[0]

    w_sh = W1[:, :D].T
    w_th = W1[:, D:2 * D].T
    w_sx = W1[:, 2 * D:2 * D + SF].T
    w_tx = W1[:, 2 * D + SF:2 * D + 2 * SF].T
    w_e = W1[:, 2 * D + 2 * SF:].T
    w2t = W2.T
    wout_t = Wout.T

    bn = 1000
    grid_n = N // bn
    S, T = pl.pallas_call(
        _node_proj_body,
        grid=(grid_n,),
        in_specs=[
            pl.BlockSpec((bn, D), lambda i: (i, 0)),
            pl.BlockSpec((bn, SF), lambda i: (i, 0)),
            pl.BlockSpec((D, width), lambda i: (0, 0)),
            pl.BlockSpec((SF, width), lambda i: (0, 0)),
            pl.BlockSpec((D, width), lambda i: (0, 0)),
            pl.BlockSpec((SF, width), lambda i: (0, 0)),
        ],
        out_specs=[
            pl.BlockSpec((bn, width // 2), lambda i: (i, 0)),
            pl.BlockSpec((bn, width // 2), lambda i: (i, 0)),
        ],
        out_shape=[
            jax.ShapeDtypeStruct((N, width // 2), jnp.int32),
            jax.ShapeDtypeStruct((N, width // 2), jnp.int32),
        ],
    )(h, x_s, w_sh, w_sx, w_th, w_tx)

    EF = edge_features.shape[1]
    wh = width // 2
    be = 2000
    Eh = E // 2
    zeros = jnp.zeros((N, D), jnp.float32)
    gather_k = _make_gather_add(Eh, width, 128)
    scatter_k = _make_scatter(N, Eh, D, H, 80)

    def edge_mlp(pre_s, pre_t, ef):
        return pl.pallas_call(
            _edge_mlp_body,
            grid=(Eh // be,),
            in_specs=[
                pl.BlockSpec((be, wh), lambda i: (i, 0)),
                pl.BlockSpec((be, wh), lambda i: (i, 0)),
                pl.BlockSpec((be, EF), lambda i: (i, 0)),
                pl.BlockSpec((EF, wh), lambda i: (0, 0)),
                pl.BlockSpec((EF, wh), lambda i: (0, 0)),
                pl.BlockSpec((wh, H), lambda i: (0, 0)),
                pl.BlockSpec((wh, H), lambda i: (0, 0)),
            ],
            out_specs=pl.BlockSpec((be, H), lambda i: (i, 0)),
            out_shape=jax.ShapeDtypeStruct((Eh, H), jnp.float32),
        )(pre_s, pre_t, ef, w_e[:, :wh], w_e[:, wh:], w2t[:wh], w2t[wh:])

    halves = []
    for lo in (0, Eh):
        halves.append({
            "src": lax.slice(src, (lo,), (lo + Eh,)),
            "dest": lax.slice(dest, (lo,), (lo + Eh,)),
            "ef": lax.slice(edge_features, (lo, 0), (lo + Eh, EF)),
        })

    pre_a = gather_k(S, T, halves[0]["src"], halves[0]["dest"])
    pre_b = gather_k(S, T, halves[1]["src"], halves[1]["dest"])

    gates_a = edge_mlp(pre_a[0], pre_a[1], halves[0]["ef"])
    gates_b = edge_mlp(pre_b[0], pre_b[1], halves[1]["ef"])

    agg_a = scatter_k(h, gates_a.reshape(Eh * H), halves[0]["src"],
                      halves[0]["dest"], zeros)
    agg_b = scatter_k(h, gates_b.reshape(Eh * H), halves[1]["src"],
                      halves[1]["dest"], zeros)

    gates = jnp.concatenate([gates_a, gates_b], axis=0)

    projected = pl.pallas_call(
        _out_proj_body,
        grid=(grid_n,),
        in_specs=[
            pl.BlockSpec((bn, D), lambda i: (i, 0)),
            pl.BlockSpec((bn, D), lambda i: (i, 0)),
            pl.BlockSpec((bn, D), lambda i: (i, 0)),
            pl.BlockSpec((bn, D), lambda i: (i, 0)),
            pl.BlockSpec((D, D), lambda i: (0, 0)),
        ],
        out_specs=pl.BlockSpec((bn, D), lambda i: (i, 0)),
        out_shape=jax.ShapeDtypeStruct((N, D), jnp.float32),
    )(agg_a[0], agg_a[1], agg_b[0], agg_b[1], wout_t)

    return projected, gates


# speedup vs baseline: 1.1203x; 1.0483x over previous
import functools

import jax
import jax.numpy as jnp
from jax import lax
from jax.experimental import pallas as pl
from jax.experimental.pallas import tpu as pltpu
from jax.experimental.pallas import tpu_sc as plsc

NC = 2
NS = 16
NW = NC * NS
LANES = 16


def _pack2(x):
    k = x.shape[1] // 2
    lo = jax.lax.bitcast_convert_type(
        x[:, :k].astype(jnp.bfloat16), jnp.uint16).astype(jnp.uint32)
    hi = jax.lax.bitcast_convert_type(
        x[:, k:].astype(jnp.bfloat16), jnp.uint16).astype(jnp.uint32)
    return jax.lax.bitcast_convert_type(lo | (hi << 16), jnp.int32)


def _unpack2(p):
    pu = jax.lax.bitcast_convert_type(p, jnp.uint32)
    lo = jax.lax.bitcast_convert_type(
        (pu & 0xFFFF).astype(jnp.uint16), jnp.bfloat16).astype(jnp.float32)
    hi = jax.lax.bitcast_convert_type(
        (pu >> 16).astype(jnp.uint16), jnp.bfloat16).astype(jnp.float32)
    return lo, hi


def _node_proj_body(h_ref, xs_ref, wsh_ref, wsx_ref, wth_ref, wtx_ref,
                    s_ref, t_ref):
    h = h_ref[...]
    xs = xs_ref[...]
    s = (jnp.dot(h, wsh_ref[...], preferred_element_type=jnp.float32)
         + jnp.dot(xs, wsx_ref[...], preferred_element_type=jnp.float32))
    t = (jnp.dot(h, wth_ref[...], preferred_element_type=jnp.float32)
         + jnp.dot(xs, wtx_ref[...], preferred_element_type=jnp.float32))
    s_ref[...] = _pack2(s)
    t_ref[...] = _pack2(t)


def _edge_mlp_body(ps_ref, pt_ref, ef_ref, w1e0_ref, w1e1_ref,
                   w2t0_ref, w2t1_ref, g_ref):
    s_lo, s_hi = _unpack2(ps_ref[...])
    t_lo, t_hi = _unpack2(pt_ref[...])
    ef = ef_ref[...]
    h0 = jnp.maximum(
        s_lo + t_lo
        + jnp.dot(ef, w1e0_ref[...], preferred_element_type=jnp.float32),
        0.0)
    h1 = jnp.maximum(
        s_hi + t_hi
        + jnp.dot(ef, w1e1_ref[...], preferred_element_type=jnp.float32),
        0.0)
    raw = (jnp.dot(h0, w2t0_ref[...], preferred_element_type=jnp.float32)
           + jnp.dot(h1, w2t1_ref[...], preferred_element_type=jnp.float32))
    g_ref[...] = jax.nn.sigmoid(raw)


def _out_proj_body(a0_ref, a1_ref, b0_ref, b1_ref, w_ref, o_ref):
    acc = (a0_ref[...] + a1_ref[...]) + (b0_ref[...] + b1_ref[...])
    o_ref[...] = jnp.dot(acc, w_ref[...],
                         preferred_element_type=jnp.float32)


def _make_gather_add(N, E, width, chunk):
    n_chunks_total = E // chunk
    full_rounds = n_chunks_total // NW
    extra = n_chunks_total - full_rounds * NW
    NB = 3
    n_groups = full_rounds // NB
    remg = full_rounds - n_groups * NB
    mesh = plsc.VectorSubcoreMesh(core_axis_name="c", subcore_axis_name="s")
    wp = width // 2

    @functools.partial(
        pl.kernel, mesh=mesh,
        out_type=[
            jax.ShapeDtypeStruct((E, wp), jnp.int32),
            jax.ShapeDtypeStruct((E, wp), jnp.int32),
        ],
        scratch_types=(
            [pltpu.VMEM((chunk,), jnp.int32)] * (2 * NB)
            + [pltpu.VMEM((chunk, wp), jnp.int32)] * (2 * NB)
            + [pltpu.VMEM_SHARED((N, wp), jnp.int32)]
            + [pltpu.SemaphoreType.DMA] * (3 * NB)
        ))
    def gather_add(s_hbm, t_hbm, src_hbm, dst_hbm, ps_hbm, pt_hbm, *scr):
        sidxs = scr[0:NB]
        didxs = scr[NB:2 * NB]
        srows = scr[2 * NB:3 * NB]
        trows = scr[3 * NB:4 * NB]
        s_sp = scr[4 * NB]
        sis = scr[4 * NB + 1:5 * NB + 1]
        sss = scr[5 * NB + 1:6 * NB + 1]
        sts = scr[6 * NB + 1:7 * NB + 1]
        wid = lax.axis_index("s") * NC + lax.axis_index("c")

        @pl.when(lax.axis_index("s") == 0)
        def _():
            pltpu.sync_copy(s_hbm, s_sp)

        plsc.subcore_barrier()

        def cbase(k):
            return pl.multiple_of((wid + k * NW) * chunk, 8)

        def idx_fetch(k, b):
            base = cbase(k)
            return (
                pltpu.async_copy(src_hbm.at[pl.ds(base, chunk)],
                                 sidxs[b], sis[b]),
                pltpu.async_copy(dst_hbm.at[pl.ds(base, chunk)],
                                 didxs[b], sis[b]),
            )

        def gath(b):
            return (
                pltpu.async_copy(s_sp.at[sidxs[b]], srows[b], sss[b]),
                pltpu.async_copy(t_hbm.at[didxs[b]], trows[b], sts[b]),
            )

        def out(k, b):
            base = cbase(k)
            return (
                pltpu.async_copy(srows[b], ps_hbm.at[pl.ds(base, chunk)],
                                 sss[b]),
                pltpu.async_copy(trows[b], pt_hbm.at[pl.ds(base, chunk)],
                                 sts[b]),
            )

        def group(k0, m):
            ids = [idx_fetch(k0 + q, q) for q in range(m)]
            gs = []
            for q in range(m):
                for cp in ids[q]:
                    cp.wait()
                gs.append(gath(q))
            outs = []
            for q in range(m):
                for cp in gs[q]:
                    cp.wait()
                outs.append(out(k0 + q, q))
            for o in outs:
                for cp in o:
                    cp.wait()

        def body(i, carry):
            group(NB * i, NB)
            return carry

        lax.fori_loop(0, n_groups, body, 0)
        if remg:
            group(n_groups * NB, remg)
        if extra:
            @pl.when(wid < extra)
            def _():
                group(full_rounds, 1)

    return gather_add


def _make_scatter(N, E, D, H, chunk):
    n_chunks_total = E // chunk
    full_rounds = n_chunks_total // NW
    extra = n_chunks_total - full_rounds * NW
    hd = D // H
    assert hd == LANES
    mesh = plsc.VectorSubcoreMesh(core_axis_name="c", subcore_axis_name="s")

    NB = 4
    n_groups = full_rounds // NB
    remg = full_rounds - n_groups * NB

    @functools.partial(
        pl.kernel, mesh=mesh,
        out_type=jax.ShapeDtypeStruct((NC, N, D), jnp.float32),
        scratch_types=(
            [pltpu.VMEM((chunk,), jnp.int32)] * (2 * NB)
            + [pltpu.VMEM((chunk, D), jnp.float32)] * NB
            + [pltpu.VMEM((chunk * H,), jnp.float32)] * NB
            + [pltpu.VMEM_SHARED((N, D), jnp.float32)]
            + [pltpu.SemaphoreType.DMA] * (4 * NB)
        ))
    def scatter(h_hbm, gates_hbm, src_hbm, dst_hbm, zeros_hbm, agg_hbm,
                *scr):
        sidxs = scr[0:NB]
        didxs = scr[NB:2 * NB]
        hrows = scr[2 * NB:3 * NB]
        grows = scr[3 * NB:4 * NB]
        agg_sp = scr[4 * NB]
        sis = scr[4 * NB + 1:5 * NB + 1]
        shs = scr[5 * NB + 1:6 * NB + 1]
        sgs = scr[6 * NB + 1:7 * NB + 1]
        sc_s = scr[7 * NB + 1:8 * NB + 1]
        cid = lax.axis_index("c")
        sid = lax.axis_index("s")
        wid = sid * NC + cid

        @pl.when(sid == 0)
        def _():
            pltpu.sync_copy(zeros_hbm, agg_sp)

        plsc.subcore_barrier()

        def cbase(k):
            return pl.multiple_of((wid + k * NW) * chunk, 8)

        def idx_fetch(k, b):
            base = cbase(k)
            return (
                pltpu.async_copy(src_hbm.at[pl.ds(base, chunk)],
                                 sidxs[b], sis[b]),
                pltpu.async_copy(dst_hbm.at[pl.ds(base, chunk)],
                                 didxs[b], sis[b]),
            )

        def gath(k, b):
            gbase = pl.multiple_of(cbase(k) * H, 8)
            return (
                pltpu.async_copy(h_hbm.at[sidxs[b]], hrows[b], shs[b]),
                pltpu.async_copy(gates_hbm.at[pl.ds(gbase, chunk * H)],
                                 grows[b], sgs[b]),
            )

        def compute_scat(b):
            hrow, grow = hrows[b], grows[b]

            def epair(p, c2):
                gv = grow[pl.ds(p * 2 * H, 2 * H)]
                for b2 in range(2):
                    e = 2 * p + b2
                    for j in range(H):
                        idxc = jnp.full((LANES,), b2 * H + j, jnp.int32)
                        gsp = gv.at[idxc].get(mode="promise_in_bounds")
                        sl = pl.ds(j * LANES, LANES)
                        hrow[e, sl] = hrow[e, sl] * gsp
                return c2

            lax.fori_loop(0, chunk // 2, epair, 0)
            return pltpu.async_copy(hrow, agg_sp.at[didxs[b]], sc_s[b],
                                    add=True)

        def group(k0, m):
            ids = [idx_fetch(k0 + q, q) for q in range(m)]
            gs = []
            for q in range(m):
                for cp in ids[q]:
                    cp.wait()
                gs.append(gath(k0 + q, q))
            scs = []
            for q in range(m):
                for cp in gs[q]:
                    cp.wait()
                scs.append(compute_scat(q))
            for cp in scs:
                cp.wait()

        def body(i, carry):
            group(NB * i, NB)
            return carry

        lax.fori_loop(0, n_groups, body, 0)
        if remg:
            group(n_groups * NB, remg)
        if extra:
            @pl.when(wid < extra)
            def _():
                group(full_rounds, 1)

        plsc.subcore_barrier()

        @pl.when(sid == 0)
        def _():
            pltpu.sync_copy(agg_sp, agg_hbm.at[cid])

    return scatter


def kernel(h, x_s, edge_index, edge_features, W1, W2, Wout):
    N, D = h.shape
    SF = x_s.shape[1]
    E = edge_index.shape[1]
    H, width = W2.shape

    src = edge_index[1]
    dest = edge_index[0]

    w_sh = W1[:, :D].T
    w_th = W1[:, D:2 * D].T
    w_sx = W1[:, 2 * D:2 * D + SF].T
    w_tx = W1[:, 2 * D + SF:2 * D + 2 * SF].T
    w_e = W1[:, 2 * D + 2 * SF:].T
    w2t = W2.T
    wout_t = Wout.T

    bn = 1000
    grid_n = N // bn
    S, T = pl.pallas_call(
        _node_proj_body,
        grid=(grid_n,),
        in_specs=[
            pl.BlockSpec((bn, D), lambda i: (i, 0)),
            pl.BlockSpec((bn, SF), lambda i: (i, 0)),
            pl.BlockSpec((D, width), lambda i: (0, 0)),
            pl.BlockSpec((SF, width), lambda i: (0, 0)),
            pl.BlockSpec((D, width), lambda i: (0, 0)),
            pl.BlockSpec((SF, width), lambda i: (0, 0)),
        ],
        out_specs=[
            pl.BlockSpec((bn, width // 2), lambda i: (i, 0)),
            pl.BlockSpec((bn, width // 2), lambda i: (i, 0)),
        ],
        out_shape=[
            jax.ShapeDtypeStruct((N, width // 2), jnp.int32),
            jax.ShapeDtypeStruct((N, width // 2), jnp.int32),
        ],
    )(h, x_s, w_sh, w_sx, w_th, w_tx)

    EF = edge_features.shape[1]
    wh = width // 2
    be = 2000
    Eh = E // 2
    zeros = jnp.zeros((N, D), jnp.float32)
    gather_k = _make_gather_add(N, Eh, width, 64)
    scatter_k = _make_scatter(N, Eh, D, H, 80)

    def edge_mlp(pre_s, pre_t, ef):
        return pl.pallas_call(
            _edge_mlp_body,
            grid=(Eh // be,),
            in_specs=[
                pl.BlockSpec((be, wh), lambda i: (i, 0)),
                pl.BlockSpec((be, wh), lambda i: (i, 0)),
                pl.BlockSpec((be, EF), lambda i: (i, 0)),
                pl.BlockSpec((EF, wh), lambda i: (0, 0)),
                pl.BlockSpec((EF, wh), lambda i: (0, 0)),
                pl.BlockSpec((wh, H), lambda i: (0, 0)),
                pl.BlockSpec((wh, H), lambda i: (0, 0)),
            ],
            out_specs=pl.BlockSpec((be, H), lambda i: (i, 0)),
            out_shape=jax.ShapeDtypeStruct((Eh, H), jnp.float32),
        )(pre_s, pre_t, ef, w_e[:, :wh], w_e[:, wh:], w2t[:wh], w2t[wh:])

    halves = []
    for lo in (0, Eh):
        halves.append({
            "src": lax.slice(src, (lo,), (lo + Eh,)),
            "dest": lax.slice(dest, (lo,), (lo + Eh,)),
            "ef": lax.slice(edge_features, (lo, 0), (lo + Eh, EF)),
        })

    pre_a = gather_k(S, T, halves[0]["src"], halves[0]["dest"])
    pre_b = gather_k(S, T, halves[1]["src"], halves[1]["dest"])

    gates_a = edge_mlp(pre_a[0], pre_a[1], halves[0]["ef"])
    gates_b = edge_mlp(pre_b[0], pre_b[1], halves[1]["ef"])

    agg_a = scatter_k(h, gates_a.reshape(Eh * H), halves[0]["src"],
                      halves[0]["dest"], zeros)
    agg_b = scatter_k(h, gates_b.reshape(Eh * H), halves[1]["src"],
                      halves[1]["dest"], zeros)

    gates = jnp.concatenate([gates_a, gates_b], axis=0)

    projected = pl.pallas_call(
        _out_proj_body,
        grid=(grid_n,),
        in_specs=[
            pl.BlockSpec((bn, D), lambda i: (i, 0)),
            pl.BlockSpec((bn, D), lambda i: (i, 0)),
            pl.BlockSpec((bn, D), lambda i: (i, 0)),
            pl.BlockSpec((bn, D), lambda i: (i, 0)),
            pl.BlockSpec((D, D), lambda i: (0, 0)),
        ],
        out_specs=pl.BlockSpec((bn, D), lambda i: (i, 0)),
        out_shape=jax.ShapeDtypeStruct((N, D), jnp.float32),
    )(agg_a[0], agg_a[1], agg_b[0], agg_b[1], wout_t)

    return projected, gates


# speedup vs baseline: 1.1273x; 1.0063x over previous
import functools

import jax
import jax.numpy as jnp
from jax import lax
from jax.experimental import pallas as pl
from jax.experimental.pallas import tpu as pltpu
from jax.experimental.pallas import tpu_sc as plsc

NC = 2
NS = 16
NW = NC * NS
LANES = 16


def _pack2(x):
    k = x.shape[1] // 2
    lo = jax.lax.bitcast_convert_type(
        x[:, :k].astype(jnp.bfloat16), jnp.uint16).astype(jnp.uint32)
    hi = jax.lax.bitcast_convert_type(
        x[:, k:].astype(jnp.bfloat16), jnp.uint16).astype(jnp.uint32)
    return jax.lax.bitcast_convert_type(lo | (hi << 16), jnp.int32)


def _unpack2(p):
    lo = jax.lax.bitcast_convert_type(p << 16, jnp.float32)
    hi = jax.lax.bitcast_convert_type(p & jnp.int32(-65536), jnp.float32)
    return lo, hi


def _node_proj_body(h_ref, xs_ref, wsh_ref, wsx_ref, wth_ref, wtx_ref,
                    s_ref, t_ref):
    h = h_ref[...]
    xs = xs_ref[...]
    s = (jnp.dot(h, wsh_ref[...], preferred_element_type=jnp.float32)
         + jnp.dot(xs, wsx_ref[...], preferred_element_type=jnp.float32))
    t = (jnp.dot(h, wth_ref[...], preferred_element_type=jnp.float32)
         + jnp.dot(xs, wtx_ref[...], preferred_element_type=jnp.float32))
    s_ref[...] = _pack2(s)
    t_ref[...] = _pack2(t)


def _edge_mlp_body(ps_ref, pt_ref, ef_ref, w1e0_ref, w1e1_ref,
                   w2t0_ref, w2t1_ref, g_ref):
    s_lo, s_hi = _unpack2(ps_ref[...])
    t_lo, t_hi = _unpack2(pt_ref[...])
    ef = ef_ref[...]
    h0 = jnp.maximum(
        s_lo + t_lo
        + jnp.dot(ef, w1e0_ref[...], preferred_element_type=jnp.float32),
        0.0)
    h1 = jnp.maximum(
        s_hi + t_hi
        + jnp.dot(ef, w1e1_ref[...], preferred_element_type=jnp.float32),
        0.0)
    raw = (jnp.dot(h0, w2t0_ref[...], preferred_element_type=jnp.float32)
           + jnp.dot(h1, w2t1_ref[...], preferred_element_type=jnp.float32))
    g_ref[...] = jax.nn.sigmoid(raw)


def _out_proj_body(a0_ref, a1_ref, b0_ref, b1_ref, w_ref, o_ref):
    acc = (a0_ref[...] + a1_ref[...]) + (b0_ref[...] + b1_ref[...])
    o_ref[...] = jnp.dot(acc, w_ref[...],
                         preferred_element_type=jnp.float32)


def _make_gather_add(N, E, width, chunk):
    n_chunks_total = E // chunk
    full_rounds = n_chunks_total // NW
    extra = n_chunks_total - full_rounds * NW
    NB = 3
    n_groups = full_rounds // NB
    remg = full_rounds - n_groups * NB
    mesh = plsc.VectorSubcoreMesh(core_axis_name="c", subcore_axis_name="s")
    wp = width // 2

    @functools.partial(
        pl.kernel, mesh=mesh,
        out_type=[
            jax.ShapeDtypeStruct((E, wp), jnp.int32),
            jax.ShapeDtypeStruct((E, wp), jnp.int32),
        ],
        scratch_types=(
            [pltpu.VMEM((chunk,), jnp.int32)] * (2 * NB)
            + [pltpu.VMEM((chunk, wp), jnp.int32)] * (2 * NB)
            + [pltpu.VMEM_SHARED((N, wp), jnp.int32)]
            + [pltpu.SemaphoreType.DMA] * (3 * NB)
        ))
    def gather_add(s_hbm, t_hbm, src_hbm, dst_hbm, ps_hbm, pt_hbm, *scr):
        sidxs = scr[0:NB]
        didxs = scr[NB:2 * NB]
        srows = scr[2 * NB:3 * NB]
        trows = scr[3 * NB:4 * NB]
        s_sp = scr[4 * NB]
        sis = scr[4 * NB + 1:5 * NB + 1]
        sss = scr[5 * NB + 1:6 * NB + 1]
        sts = scr[6 * NB + 1:7 * NB + 1]
        wid = lax.axis_index("s") * NC + lax.axis_index("c")

        @pl.when(lax.axis_index("s") == 0)
        def _():
            pltpu.sync_copy(s_hbm, s_sp)

        plsc.subcore_barrier()

        def cbase(k):
            return pl.multiple_of((wid + k * NW) * chunk, 8)

        def idx_fetch(k, b):
            base = cbase(k)
            return (
                pltpu.async_copy(src_hbm.at[pl.ds(base, chunk)],
                                 sidxs[b], sis[b]),
                pltpu.async_copy(dst_hbm.at[pl.ds(base, chunk)],
                                 didxs[b], sis[b]),
            )

        def gath(b):
            return (
                pltpu.async_copy(s_sp.at[sidxs[b]], srows[b], sss[b]),
                pltpu.async_copy(t_hbm.at[didxs[b]], trows[b], sts[b]),
            )

        def out(k, b):
            base = cbase(k)
            return (
                pltpu.async_copy(srows[b], ps_hbm.at[pl.ds(base, chunk)],
                                 sss[b]),
                pltpu.async_copy(trows[b], pt_hbm.at[pl.ds(base, chunk)],
                                 sts[b]),
            )

        def group(k0, m):
            ids = [idx_fetch(k0 + q, q) for q in range(m)]
            gs = []
            for q in range(m):
                for cp in ids[q]:
                    cp.wait()
                gs.append(gath(q))
            outs = []
            for q in range(m):
                for cp in gs[q]:
                    cp.wait()
                outs.append(out(k0 + q, q))
            for o in outs:
                for cp in o:
                    cp.wait()

        def body(i, carry):
            group(NB * i, NB)
            return carry

        lax.fori_loop(0, n_groups, body, 0)
        if remg:
            group(n_groups * NB, remg)
        if extra:
            @pl.when(wid < extra)
            def _():
                group(full_rounds, 1)

    return gather_add


def _make_scatter(N, E, D, H, chunk):
    n_chunks_total = E // chunk
    full_rounds = n_chunks_total // NW
    extra = n_chunks_total - full_rounds * NW
    hd = D // H
    assert hd == LANES
    mesh = plsc.VectorSubcoreMesh(core_axis_name="c", subcore_axis_name="s")

    NB = 4
    n_groups = full_rounds // NB
    remg = full_rounds - n_groups * NB

    @functools.partial(
        pl.kernel, mesh=mesh,
        out_type=jax.ShapeDtypeStruct((NC, N, D), jnp.float32),
        scratch_types=(
            [pltpu.VMEM((chunk,), jnp.int32)] * (2 * NB)
            + [pltpu.VMEM((chunk, D), jnp.float32)] * NB
            + [pltpu.VMEM((chunk * H,), jnp.float32)] * NB
            + [pltpu.VMEM_SHARED((N, D), jnp.float32)]
            + [pltpu.SemaphoreType.DMA] * (4 * NB)
        ))
    def scatter(h_hbm, gates_hbm, src_hbm, dst_hbm, zeros_hbm, agg_hbm,
                *scr):
        sidxs = scr[0:NB]
        didxs = scr[NB:2 * NB]
        hrows = scr[2 * NB:3 * NB]
        grows = scr[3 * NB:4 * NB]
        agg_sp = scr[4 * NB]
        sis = scr[4 * NB + 1:5 * NB + 1]
        shs = scr[5 * NB + 1:6 * NB + 1]
        sgs = scr[6 * NB + 1:7 * NB + 1]
        sc_s = scr[7 * NB + 1:8 * NB + 1]
        cid = lax.axis_index("c")
        sid = lax.axis_index("s")
        wid = sid * NC + cid

        @pl.when(sid == 0)
        def _():
            pltpu.sync_copy(zeros_hbm, agg_sp)

        plsc.subcore_barrier()

        def cbase(k):
            return pl.multiple_of((wid + k * NW) * chunk, 8)

        def idx_fetch(k, b):
            base = cbase(k)
            return (
                pltpu.async_copy(src_hbm.at[pl.ds(base, chunk)],
                                 sidxs[b], sis[b]),
                pltpu.async_copy(dst_hbm.at[pl.ds(base, chunk)],
                                 didxs[b], sis[b]),
            )

        def gath(k, b):
            gbase = pl.multiple_of(cbase(k) * H, 8)
            return (
                pltpu.async_copy(h_hbm.at[sidxs[b]], hrows[b], shs[b]),
                pltpu.async_copy(gates_hbm.at[pl.ds(gbase, chunk * H)],
                                 grows[b], sgs[b]),
            )

        def compute_scat(b):
            hrow, grow = hrows[b], grows[b]

            def epair(p, c2):
                gv = grow[pl.ds(p * 2 * H, 2 * H)]
                for b2 in range(2):
                    e = 2 * p + b2
                    for j in range(H):
                        idxc = jnp.full((LANES,), b2 * H + j, jnp.int32)
                        gsp = gv.at[idxc].get(mode="promise_in_bounds")
                        sl = pl.ds(j * LANES, LANES)
                        hrow[e, sl] = hrow[e, sl] * gsp
                return c2

            lax.fori_loop(0, chunk // 2, epair, 0)
            return pltpu.async_copy(hrow, agg_sp.at[didxs[b]], sc_s[b],
                                    add=True)

        def group(k0, m):
            ids = [idx_fetch(k0 + q, q) for q in range(m)]
            gs = []
            for q in range(m):
                for cp in ids[q]:
                    cp.wait()
                gs.append(gath(k0 + q, q))
            scs = []
            for q in range(m):
                for cp in gs[q]:
                    cp.wait()
                scs.append(compute_scat(q))
            for cp in scs:
                cp.wait()

        def body(i, carry):
            group(NB * i, NB)
            return carry

        lax.fori_loop(0, n_groups, body, 0)
        if remg:
            group(n_groups * NB, remg)
        if extra:
            @pl.when(wid < extra)
            def _():
                group(full_rounds, 1)

        plsc.subcore_barrier()

        @pl.when(sid == 0)
        def _():
            pltpu.sync_copy(agg_sp, agg_hbm.at[cid])

    return scatter


def kernel(h, x_s, edge_index, edge_features, W1, W2, Wout):
    N, D = h.shape
    SF = x_s.shape[1]
    E = edge_index.shape[1]
    H, width = W2.shape

    src = edge_index[1]
    dest = edge_index[0]

    w_sh = W1[:, :D].T
    w_th = W1[:, D:2 * D].T
    w_sx = W1[:, 2 * D:2 * D + SF].T
    w_tx = W1[:, 2 * D + SF:2 * D + 2 * SF].T
    w_e = W1[:, 2 * D + 2 * SF:].T
    w2t = W2.T
    wout_t = Wout.T

    bn = 1000
    grid_n = N // bn
    S, T = pl.pallas_call(
        _node_proj_body,
        grid=(grid_n,),
        in_specs=[
            pl.BlockSpec((bn, D), lambda i: (i, 0)),
            pl.BlockSpec((bn, SF), lambda i: (i, 0)),
            pl.BlockSpec((D, width), lambda i: (0, 0)),
            pl.BlockSpec((SF, width), lambda i: (0, 0)),
            pl.BlockSpec((D, width), lambda i: (0, 0)),
            pl.BlockSpec((SF, width), lambda i: (0, 0)),
        ],
        out_specs=[
            pl.BlockSpec((bn, width // 2), lambda i: (i, 0)),
            pl.BlockSpec((bn, width // 2), lambda i: (i, 0)),
        ],
        out_shape=[
            jax.ShapeDtypeStruct((N, width // 2), jnp.int32),
            jax.ShapeDtypeStruct((N, width // 2), jnp.int32),
        ],
    )(h, x_s, w_sh, w_sx, w_th, w_tx)

    EF = edge_features.shape[1]
    wh = width // 2
    be = 2000
    Eh = E // 2
    zeros = jnp.zeros((N, D), jnp.float32)
    gather_k = _make_gather_add(N, Eh, width, 64)
    scatter_k = _make_scatter(N, Eh, D, H, 80)

    def edge_mlp(pre_s, pre_t, ef):
        return pl.pallas_call(
            _edge_mlp_body,
            grid=(Eh // be,),
            in_specs=[
                pl.BlockSpec((be, wh), lambda i: (i, 0)),
                pl.BlockSpec((be, wh), lambda i: (i, 0)),
                pl.BlockSpec((be, EF), lambda i: (i, 0)),
                pl.BlockSpec((EF, wh), lambda i: (0, 0)),
                pl.BlockSpec((EF, wh), lambda i: (0, 0)),
                pl.BlockSpec((wh, H), lambda i: (0, 0)),
                pl.BlockSpec((wh, H), lambda i: (0, 0)),
            ],
            out_specs=pl.BlockSpec((be, H), lambda i: (i, 0)),
            out_shape=jax.ShapeDtypeStruct((Eh, H), jnp.float32),
        )(pre_s, pre_t, ef, w_e[:, :wh], w_e[:, wh:], w2t[:wh], w2t[wh:])

    halves = []
    for lo in (0, Eh):
        halves.append({
            "src": lax.slice(src, (lo,), (lo + Eh,)),
            "dest": lax.slice(dest, (lo,), (lo + Eh,)),
            "ef": lax.slice(edge_features, (lo, 0), (lo + Eh, EF)),
        })

    pre_a = gather_k(S, T, halves[0]["src"], halves[0]["dest"])
    pre_b = gather_k(S, T, halves[1]["src"], halves[1]["dest"])

    gates_a = edge_mlp(pre_a[0], pre_a[1], halves[0]["ef"])
    gates_b = edge_mlp(pre_b[0], pre_b[1], halves[1]["ef"])

    agg_a = scatter_k(h, gates_a.reshape(Eh * H), halves[0]["src"],
                      halves[0]["dest"], zeros)
    agg_b = scatter_k(h, gates_b.reshape(Eh * H), halves[1]["src"],
                      halves[1]["dest"], zeros)

    gates = jnp.concatenate([gates_a, gates_b], axis=0)

    projected = pl.pallas_call(
        _out_proj_body,
        grid=(grid_n,),
        in_specs=[
            pl.BlockSpec((bn, D), lambda i: (i, 0)),
            pl.BlockSpec((bn, D), lambda i: (i, 0)),
            pl.BlockSpec((bn, D), lambda i: (i, 0)),
            pl.BlockSpec((bn, D), lambda i: (i, 0)),
            pl.BlockSpec((D, D), lambda i: (0, 0)),
        ],
        out_specs=pl.BlockSpec((bn, D), lambda i: (i, 0)),
        out_shape=jax.ShapeDtypeStruct((N, D), jnp.float32),
    )(agg_a[0], agg_a[1], agg_b[0], agg_b[1], wout_t)

    return projected, gates
